# Initial kernel scaffold; baseline (speedup 1.0000x reference)
import jax
import jax.numpy as jnp
from jax.experimental import pallas as pl


def kernel(x, edge_index, W1_root, W1_neigh, b1, W2_root, W2_neigh, b2):
    raise NotImplementedError("write your pallas kernel here")


# speedup vs baseline: 10.9614x; 10.9614x over previous
import functools

import jax
import jax.numpy as jnp
from jax import lax
from jax.experimental import pallas as pl
from jax.experimental.pallas import tpu as pltpu
from jax.experimental.pallas import tpu_sc as plsc

N = 10000
E = 320000
D_IN = 128
D_HID = 16
D_OUT = 40

NC = 2
NS = 16
NW = NC * NS
CH = 128
NCHUNK = 80
EPT = NCHUNK * CH
E_PAD = EPT * NW
N_PAD = 10240
RPC = N_PAD // NS
DUMMY = N_PAD - 1

_f32 = jnp.float32


def _make_sc_agg(with_deg: bool):
  out_type = [jax.ShapeDtypeStruct((NC * N_PAD, D_HID), _f32)]
  scratch = [
      pltpu.VMEM((NCHUNK, CH), jnp.int32),
      pltpu.VMEM((NCHUNK, CH), jnp.int32),
      pltpu.VMEM((CH, D_HID), _f32),
      pltpu.VMEM((CH, D_HID), _f32),
      pltpu.VMEM_SHARED((N_PAD, D_HID), _f32),
      pltpu.SemaphoreType.DMA,
  ]
  if with_deg:
    out_type.append(jax.ShapeDtypeStruct((NC * N_PAD, D_HID), _f32))
    scratch.append(pltpu.VMEM((CH, D_HID), _f32))
    scratch.append(pltpu.VMEM_SHARED((N_PAD, D_HID), _f32))

  mesh = plsc.VectorSubcoreMesh(core_axis_name="c", subcore_axis_name="s")

  def body(src_hbm, dst_hbm, feat_hbm, zeros_hbm, ones_hbm,
           agg_out, deg_out, src_v, dst_v, rows_v, zeros_v, agg_sh, sem,
           ones_v=None, deg_sh=None):
    c = lax.axis_index("c")
    s = lax.axis_index("s")
    wid = c * NS + s

    pltpu.sync_copy(src_hbm.at[pl.ds(wid * NCHUNK, NCHUNK)], src_v)
    pltpu.sync_copy(dst_hbm.at[pl.ds(wid * NCHUNK, NCHUNK)], dst_v)
    pltpu.sync_copy(zeros_hbm, zeros_v)
    if with_deg:
      pltpu.sync_copy(ones_hbm, ones_v)

    for j in range(RPC // CH):
      off = s * RPC + j * CH
      pltpu.sync_copy(zeros_v, agg_sh.at[pl.ds(off, CH)])
      if with_deg:
        pltpu.sync_copy(zeros_v, deg_sh.at[pl.ds(off, CH)])
    plsc.subcore_barrier()

    def step(j, carry):
      pltpu.async_copy(feat_hbm.at[src_v.at[j]], rows_v, sem).wait()
      pltpu.sync_copy(rows_v, agg_sh.at[dst_v.at[j]], add=True)
      if with_deg:
        pltpu.sync_copy(ones_v, deg_sh.at[dst_v.at[j]], add=True)
      return carry

    lax.fori_loop(0, NCHUNK, step, 0)
    plsc.subcore_barrier()

    dst_off = c * N_PAD + s * RPC
    pltpu.sync_copy(agg_sh.at[pl.ds(s * RPC, RPC)],
                    agg_out.at[pl.ds(dst_off, RPC)])
    if with_deg:
      pltpu.sync_copy(deg_sh.at[pl.ds(s * RPC, RPC)],
                      deg_out.at[pl.ds(dst_off, RPC)])

  if with_deg:
    def body_deg(src_hbm, dst_hbm, feat_hbm, zeros_hbm, ones_hbm,
                 agg_out, deg_out, src_v, dst_v, rows_v, zeros_v, agg_sh,
                 sem, ones_v, deg_sh):
      body(src_hbm, dst_hbm, feat_hbm, zeros_hbm, ones_hbm,
           agg_out, deg_out, src_v, dst_v, rows_v, zeros_v, agg_sh, sem,
           ones_v=ones_v, deg_sh=deg_sh)
    fn = body_deg
  else:
    def body_nodeg(src_hbm, dst_hbm, feat_hbm, zeros_hbm, ones_hbm,
                   agg_out, src_v, dst_v, rows_v, zeros_v, agg_sh, sem):
      body(src_hbm, dst_hbm, feat_hbm, zeros_hbm, ones_hbm,
           agg_out, None, src_v, dst_v, rows_v, zeros_v, agg_sh, sem)
    fn = body_nodeg

  return pl.kernel(
      fn, out_type=out_type, mesh=mesh, scratch_types=scratch,
      compiler_params=pltpu.CompilerParams(use_tc_tiling_on_sc=False))


_sc_agg_deg = _make_sc_agg(True)
_sc_agg = _make_sc_agg(False)


def _l1_proj_body(x_ref, wn_ref, wr_ref, p1_ref, r1_ref):
  x = x_ref[...]
  p1_ref[:N] = jnp.dot(x, wn_ref[...], preferred_element_type=_f32)
  p1_ref[N:] = jnp.zeros((N_PAD - N, D_HID), _f32)
  r1_ref[:N] = jnp.dot(x, wr_ref[...], preferred_element_type=_f32)
  r1_ref[N:] = jnp.zeros((N_PAD - N, D_HID), _f32)


def _l1_combine_body(r1_ref, agg_ref, deg_ref, b1_ref, h_ref, inv_ref):
  agg = agg_ref[0] + agg_ref[1]
  inv = 1.0 / jnp.maximum(deg_ref[0] + deg_ref[1], 1.0)
  h_ref[...] = jnp.maximum(r1_ref[...] + agg * inv + b1_ref[...], 0.0)
  inv_ref[...] = inv


def _l2_out_body(h_ref, agg_ref, inv_ref, wr_ref, wn_ref, b2_ref, o_ref):
  mean2 = (agg_ref[0, :N] + agg_ref[1, :N]) * inv_ref[:N]
  z = (jnp.dot(h_ref[:N], wr_ref[...], preferred_element_type=_f32)
       + jnp.dot(mean2, wn_ref[...], preferred_element_type=_f32)
       + b2_ref[...])
  z = z - jnp.max(z, axis=1, keepdims=True)
  o_ref[...] = z - jnp.log(jnp.sum(jnp.exp(z), axis=1, keepdims=True))


def kernel(x, edge_index, W1_root, W1_neigh, b1, W2_root, W2_neigh, b2):
  src = edge_index[0]
  dst = edge_index[1]
  pad = E_PAD - E
  src_p = jnp.concatenate([src, jnp.full((pad,), DUMMY, jnp.int32)])
  dst_p = jnp.concatenate([dst, jnp.full((pad,), DUMMY, jnp.int32)])
  src_p = src_p.reshape(NW * NCHUNK, CH)
  dst_p = dst_p.reshape(NW * NCHUNK, CH)
  zeros_c = jnp.zeros((CH, D_HID), _f32)
  ones_c = jnp.ones((CH, D_HID), _f32)

  p1, r1 = pl.pallas_call(
      _l1_proj_body,
      out_shape=[jax.ShapeDtypeStruct((N_PAD, D_HID), _f32),
                 jax.ShapeDtypeStruct((N_PAD, D_HID), _f32)],
  )(x, W1_neigh, W1_root)

  agg1, deg = _sc_agg_deg(src_p, dst_p, p1, zeros_c, ones_c)
  agg1 = agg1.reshape(NC, N_PAD, D_HID)
  deg = deg.reshape(NC, N_PAD, D_HID)

  h, invdeg = pl.pallas_call(
      _l1_combine_body,
      out_shape=[jax.ShapeDtypeStruct((N_PAD, D_HID), _f32),
                 jax.ShapeDtypeStruct((N_PAD, D_HID), _f32)],
  )(r1, agg1, deg, jnp.broadcast_to(b1, (1, D_HID)))

  agg2 = _sc_agg(src_p, dst_p, h, zeros_c, ones_c)[0]
  agg2 = agg2.reshape(NC, N_PAD, D_HID)

  out = pl.pallas_call(
      _l2_out_body,
      out_shape=jax.ShapeDtypeStruct((N, D_OUT), _f32),
  )(h, agg2, invdeg, W2_root, W2_neigh, jnp.broadcast_to(b2, (1, D_OUT)))
  return out


# speedup vs baseline: 15.3504x; 1.4004x over previous
import functools

import jax
import jax.numpy as jnp
from jax import lax
from jax.experimental import pallas as pl
from jax.experimental.pallas import tpu as pltpu
from jax.experimental.pallas import tpu_sc as plsc

N = 10000
E = 320000
D_IN = 128
D_HID = 16
D_OUT = 40

NC = 2
NS = 16
NW = NC * NS
CH = 128
NCHUNK = 80
EPT = NCHUNK * CH
E_PAD = EPT * NW
N_PAD = 10240
RPC = N_PAD // NS
DUMMY = N_PAD - 1

_f32 = jnp.float32


NB = 4


def _make_sc_agg(with_deg: bool):
  out_type = [jax.ShapeDtypeStruct((NC * N_PAD, D_HID), _f32)]
  if with_deg:
    out_type.append(jax.ShapeDtypeStruct((NC * N_PAD, D_HID), _f32))
  scratch = [
      pltpu.VMEM((NCHUNK, CH), jnp.int32),
      pltpu.VMEM((NCHUNK, CH), jnp.int32),
  ]
  scratch += [pltpu.VMEM((CH, D_HID), _f32) for _ in range(NB)]
  scratch += [
      pltpu.VMEM((CH, D_HID), _f32),
      pltpu.VMEM_SHARED((N_PAD, D_HID), _f32),
  ]
  scratch += [pltpu.SemaphoreType.DMA for _ in range(NB)]
  if with_deg:
    scratch.append(pltpu.VMEM((CH, D_HID), _f32))
    scratch.append(pltpu.VMEM_SHARED((N_PAD, D_HID), _f32))

  mesh = plsc.VectorSubcoreMesh(core_axis_name="c", subcore_axis_name="s")

  def fn(src_hbm, dst_hbm, feat_hbm, zeros_hbm, ones_hbm, *rest):
    if with_deg:
      agg_out, deg_out = rest[0], rest[1]
      sc = rest[2:]
    else:
      agg_out, deg_out = rest[0], None
      sc = rest[1:]
    src_v, dst_v = sc[0], sc[1]
    rows = sc[2:2 + NB]
    zeros_v = sc[2 + NB]
    agg_sh = sc[3 + NB]
    gsems = sc[4 + NB:4 + 2 * NB]
    if with_deg:
      ones_v, deg_sh = sc[4 + 2 * NB], sc[5 + 2 * NB]

    c = lax.axis_index("c")
    s = lax.axis_index("s")
    wid = c * NS + s

    pltpu.sync_copy(src_hbm.at[pl.ds(wid * NCHUNK, NCHUNK)], src_v)
    pltpu.sync_copy(dst_hbm.at[pl.ds(wid * NCHUNK, NCHUNK)], dst_v)
    pltpu.sync_copy(zeros_hbm, zeros_v)
    if with_deg:
      pltpu.sync_copy(ones_hbm, ones_v)

    for b in range(NB):
      pltpu.async_copy(feat_hbm.at[src_v.at[b]], rows[b], gsems[b])

    for j in range(RPC // CH):
      off = s * RPC + j * CH
      pltpu.sync_copy(zeros_v, agg_sh.at[pl.ds(off, CH)])
      if with_deg:
        pltpu.sync_copy(zeros_v, deg_sh.at[pl.ds(off, CH)])
    plsc.subcore_barrier()

    @pl.loop(0, NCHUNK, step=NB)
    def _chunk_group(j0):
      for b in range(NB):
        j = j0 + b
        pltpu.make_async_copy(feat_hbm.at[src_v.at[j]], rows[b],
                              gsems[b]).wait()
        pltpu.sync_copy(rows[b], agg_sh.at[dst_v.at[j]], add=True)
        if with_deg:
          pltpu.sync_copy(ones_v, deg_sh.at[dst_v.at[j]], add=True)

        @pl.when(j + NB < NCHUNK)
        def _prefetch():
          pltpu.async_copy(feat_hbm.at[src_v.at[j + NB]], rows[b], gsems[b])

    plsc.subcore_barrier()

    dst_off = c * N_PAD + s * RPC
    pltpu.sync_copy(agg_sh.at[pl.ds(s * RPC, RPC)],
                    agg_out.at[pl.ds(dst_off, RPC)])
    if with_deg:
      pltpu.sync_copy(deg_sh.at[pl.ds(s * RPC, RPC)],
                      deg_out.at[pl.ds(dst_off, RPC)])

  return pl.kernel(
      fn, out_type=out_type, mesh=mesh, scratch_types=scratch,
      compiler_params=pltpu.CompilerParams(use_tc_tiling_on_sc=False))


_sc_agg_deg = _make_sc_agg(True)
_sc_agg = _make_sc_agg(False)


def _l1_proj_body(x_ref, wn_ref, wr_ref, p1_ref, r1_ref):
  x = x_ref[...]
  p1_ref[:N] = jnp.dot(x, wn_ref[...], preferred_element_type=_f32)
  p1_ref[N:] = jnp.zeros((N_PAD - N, D_HID), _f32)
  r1_ref[:N] = jnp.dot(x, wr_ref[...], preferred_element_type=_f32)
  r1_ref[N:] = jnp.zeros((N_PAD - N, D_HID), _f32)


def _l1_combine_body(r1_ref, agg_ref, deg_ref, b1_ref, h_ref, inv_ref):
  agg = agg_ref[0] + agg_ref[1]
  inv = 1.0 / jnp.maximum(deg_ref[0] + deg_ref[1], 1.0)
  h_ref[...] = jnp.maximum(r1_ref[...] + agg * inv + b1_ref[...], 0.0)
  inv_ref[...] = inv


def _l2_out_body(h_ref, agg_ref, inv_ref, wr_ref, wn_ref, b2_ref, o_ref):
  mean2 = (agg_ref[0, :N] + agg_ref[1, :N]) * inv_ref[:N]
  z = (jnp.dot(h_ref[:N], wr_ref[...], preferred_element_type=_f32)
       + jnp.dot(mean2, wn_ref[...], preferred_element_type=_f32)
       + b2_ref[...])
  z = z - jnp.max(z, axis=1, keepdims=True)
  o_ref[...] = z - jnp.log(jnp.sum(jnp.exp(z), axis=1, keepdims=True))


def kernel(x, edge_index, W1_root, W1_neigh, b1, W2_root, W2_neigh, b2):
  src = edge_index[0]
  dst = edge_index[1]
  pad = E_PAD - E
  src_p = jnp.concatenate([src, jnp.full((pad,), DUMMY, jnp.int32)])
  dst_p = jnp.concatenate([dst, jnp.full((pad,), DUMMY, jnp.int32)])
  src_p = src_p.reshape(NW * NCHUNK, CH)
  dst_p = dst_p.reshape(NW * NCHUNK, CH)
  zeros_c = jnp.zeros((CH, D_HID), _f32)
  ones_c = jnp.ones((CH, D_HID), _f32)

  p1, r1 = pl.pallas_call(
      _l1_proj_body,
      out_shape=[jax.ShapeDtypeStruct((N_PAD, D_HID), _f32),
                 jax.ShapeDtypeStruct((N_PAD, D_HID), _f32)],
  )(x, W1_neigh, W1_root)

  agg1, deg = _sc_agg_deg(src_p, dst_p, p1, zeros_c, ones_c)
  agg1 = agg1.reshape(NC, N_PAD, D_HID)
  deg = deg.reshape(NC, N_PAD, D_HID)

  h, invdeg = pl.pallas_call(
      _l1_combine_body,
      out_shape=[jax.ShapeDtypeStruct((N_PAD, D_HID), _f32),
                 jax.ShapeDtypeStruct((N_PAD, D_HID), _f32)],
  )(r1, agg1, deg, jnp.broadcast_to(b1, (1, D_HID)))

  agg2 = _sc_agg(src_p, dst_p, h, zeros_c, ones_c)[0]
  agg2 = agg2.reshape(NC, N_PAD, D_HID)

  out = pl.pallas_call(
      _l2_out_body,
      out_shape=jax.ShapeDtypeStruct((N, D_OUT), _f32),
  )(h, agg2, invdeg, W2_root, W2_neigh, jnp.broadcast_to(b2, (1, D_OUT)))
  return out


# speedup vs baseline: 19.8146x; 1.2908x over previous
import functools

import jax
import jax.numpy as jnp
from jax import lax
from jax.experimental import pallas as pl
from jax.experimental.pallas import tpu as pltpu
from jax.experimental.pallas import tpu_sc as plsc

N = 10000
E = 320000
D_IN = 128
D_HID = 16
D_OUT = 40

NC = 2
NS = 16
NW = NC * NS
CH = 128
NCHUNK = 80
EPT = NCHUNK * CH
E_PAD = EPT * NW
N_PAD = 10240
RPC = N_PAD // NS
DUMMY = N_PAD - 1

_f32 = jnp.float32


NB = 4
HALF = N_PAD // NC


def _make_sc_agg(with_deg: bool):
  out_type = [jax.ShapeDtypeStruct((NC * N_PAD, D_HID), _f32)]
  if with_deg:
    out_type.append(jax.ShapeDtypeStruct((NC * N_PAD, D_HID), _f32))
  scratch = [
      pltpu.VMEM((NCHUNK, CH), jnp.int32),
      pltpu.VMEM((NCHUNK, CH), jnp.int32),
  ]
  scratch += [pltpu.VMEM((CH, D_HID), _f32) for _ in range(NB)]
  scratch += [
      pltpu.VMEM((CH, D_HID), _f32),
      pltpu.VMEM_SHARED((N_PAD, D_HID), _f32),
  ]
  scratch += [pltpu.SemaphoreType.DMA for _ in range(NB)]
  if with_deg:
    scratch.append(pltpu.VMEM((CH, D_HID), _f32))
    scratch.append(pltpu.VMEM_SHARED((N_PAD, D_HID), _f32))

  mesh = plsc.VectorSubcoreMesh(core_axis_name="c", subcore_axis_name="s")

  def fn(src_hbm, dst_hbm, feat_hbm, zeros_hbm, ones_hbm, *rest):
    if with_deg:
      agg_out, deg_out = rest[0], rest[1]
      sc = rest[2:]
    else:
      agg_out, deg_out = rest[0], None
      sc = rest[1:]
    src_v, dst_v = sc[0], sc[1]
    rows = sc[2:2 + NB]
    zeros_v = sc[2 + NB]
    agg_sh = sc[3 + NB]
    gsems = sc[4 + NB:4 + 2 * NB]
    if with_deg:
      ones_v, deg_sh = sc[4 + 2 * NB], sc[5 + 2 * NB]

    c = lax.axis_index("c")
    s = lax.axis_index("s")
    wid = c * NS + s

    pltpu.sync_copy(src_hbm.at[pl.ds(wid * NCHUNK, NCHUNK)], src_v)
    pltpu.sync_copy(dst_hbm.at[pl.ds(wid * NCHUNK, NCHUNK)], dst_v)
    pltpu.sync_copy(zeros_hbm, zeros_v)
    if with_deg:
      pltpu.sync_copy(ones_hbm, ones_v)

    for b in range(NB):
      pltpu.async_copy(feat_hbm.at[src_v.at[b]], rows[b], gsems[b])

    for j in range(RPC // CH):
      off = s * RPC + j * CH
      pltpu.sync_copy(zeros_v, agg_sh.at[pl.ds(off, CH)])
      if with_deg:
        pltpu.sync_copy(zeros_v, deg_sh.at[pl.ds(off, CH)])
    plsc.subcore_barrier()

    @pl.loop(0, NCHUNK, step=NB)
    def _chunk_group(j0):
      for b in range(NB):
        j = j0 + b
        pltpu.make_async_copy(feat_hbm.at[src_v.at[j]], rows[b],
                              gsems[b]).wait()
        pltpu.sync_copy(rows[b], agg_sh.at[dst_v.at[j]], add=True)
        if with_deg:
          pltpu.sync_copy(ones_v, deg_sh.at[dst_v.at[j]], add=True)

        @pl.when(j + NB < NCHUNK)
        def _prefetch():
          pltpu.async_copy(feat_hbm.at[src_v.at[j + NB]], rows[b], gsems[b])

    plsc.subcore_barrier()

    dst_off = c * N_PAD + s * RPC
    pltpu.sync_copy(agg_sh.at[pl.ds(s * RPC, RPC)],
                    agg_out.at[pl.ds(dst_off, RPC)])
    if with_deg:
      pltpu.sync_copy(deg_sh.at[pl.ds(s * RPC, RPC)],
                      deg_out.at[pl.ds(dst_off, RPC)])

  return pl.kernel(
      fn, out_type=out_type, mesh=mesh, scratch_types=scratch,
      compiler_params=pltpu.CompilerParams(use_tc_tiling_on_sc=False))


_sc_agg_deg = _make_sc_agg(True)


def _make_sc_pass2():
  out_type = [
      jax.ShapeDtypeStruct((NC * N_PAD, D_HID), _f32),
      jax.ShapeDtypeStruct((N_PAD, D_HID), _f32),
  ]
  scratch = [
      pltpu.VMEM((NCHUNK, CH), jnp.int32),
      pltpu.VMEM((NCHUNK, CH), jnp.int32),
  ]
  scratch += [pltpu.VMEM((CH, D_HID), _f32) for _ in range(NB)]
  scratch += [
      pltpu.VMEM((CH, D_HID), _f32),
      pltpu.VMEM((RPC, D_HID), _f32),
      pltpu.VMEM((RPC, D_HID), _f32),
      pltpu.VMEM((RPC, D_HID), _f32),
      pltpu.VMEM((RPC, D_HID), _f32),
      pltpu.VMEM((RPC, D_HID), _f32),
      pltpu.VMEM_SHARED((N_PAD, D_HID), _f32),
      pltpu.VMEM_SHARED((N_PAD, D_HID), _f32),
  ]
  scratch += [pltpu.SemaphoreType.DMA for _ in range(NB)]

  mesh = plsc.VectorSubcoreMesh(core_axis_name="c", subcore_axis_name="s")

  def fn(src_hbm, dst_hbm, r1b_hbm, agg1_hbm, deg_hbm, zeros_hbm,
         agg2_out, h_out, src_v, dst_v, *sc):
    rows = sc[0:NB]
    zeros_v = sc[NB]
    hrow_v, a0_v, a1_v, d0_v, d1_v = sc[NB + 1:NB + 6]
    h_sh = sc[NB + 6]
    agg2_sh = sc[NB + 7]
    gsems = sc[NB + 8:NB + 8 + NB]

    c = lax.axis_index("c")
    s = lax.axis_index("s")
    wid = c * NS + s
    base = s * RPC

    pltpu.sync_copy(src_hbm.at[pl.ds(wid * NCHUNK, NCHUNK)], src_v)
    pltpu.sync_copy(dst_hbm.at[pl.ds(wid * NCHUNK, NCHUNK)], dst_v)
    pltpu.sync_copy(zeros_hbm, zeros_v)
    pltpu.sync_copy(r1b_hbm.at[pl.ds(base, RPC)], hrow_v)
    pltpu.sync_copy(agg1_hbm.at[pl.ds(base, RPC)], a0_v)
    pltpu.sync_copy(agg1_hbm.at[pl.ds(N_PAD + base, RPC)], a1_v)
    pltpu.sync_copy(deg_hbm.at[pl.ds(base, RPC)], d0_v)
    pltpu.sync_copy(deg_hbm.at[pl.ds(N_PAD + base, RPC)], d1_v)

    @pl.loop(0, RPC)
    def _hrow(i):
      mean1 = (a0_v[i] + a1_v[i]) / jnp.maximum(d0_v[i] + d1_v[i], 1.0)
      hrow_v[i] = jnp.maximum(hrow_v[i] + mean1, 0.0)

    pltpu.sync_copy(hrow_v, h_sh.at[pl.ds(base, RPC)])

    @pl.when(s // (NS // NC) == c)
    def _publish():
      pltpu.sync_copy(hrow_v, h_out.at[pl.ds(base, RPC)])

    for j in range(RPC // CH):
      pltpu.sync_copy(zeros_v, agg2_sh.at[pl.ds(base + j * CH, CH)])
    plsc.subcore_barrier()

    for b in range(NB):
      pltpu.async_copy(h_sh.at[src_v.at[b]], rows[b], gsems[b])

    @pl.loop(0, NCHUNK, step=NB)
    def _chunk_group(j0):
      for b in range(NB):
        j = j0 + b
        pltpu.make_async_copy(h_sh.at[src_v.at[j]], rows[b], gsems[b]).wait()
        pltpu.sync_copy(rows[b], agg2_sh.at[dst_v.at[j]], add=True)

        @pl.when(j + NB < NCHUNK)
        def _prefetch():
          pltpu.async_copy(h_sh.at[src_v.at[j + NB]], rows[b], gsems[b])

    plsc.subcore_barrier()
    pltpu.sync_copy(agg2_sh.at[pl.ds(base, RPC)],
                    agg2_out.at[pl.ds(c * N_PAD + base, RPC)])

  return pl.kernel(
      fn, out_type=out_type, mesh=mesh, scratch_types=scratch,
      compiler_params=pltpu.CompilerParams(use_tc_tiling_on_sc=False))


_sc_pass2 = _make_sc_pass2()


def _l1_proj_body(x_ref, wn_ref, wr_ref, b1_ref, p1_ref, r1b_ref):
  x = x_ref[...]
  p1_ref[:N] = jnp.dot(x, wn_ref[...], preferred_element_type=_f32)
  p1_ref[N:] = jnp.zeros((N_PAD - N, D_HID), _f32)
  r1b_ref[:N] = (jnp.dot(x, wr_ref[...], preferred_element_type=_f32)
                 + b1_ref[...])
  r1b_ref[N:] = jnp.zeros((N_PAD - N, D_HID), _f32)


def _l2_out_body(h_ref, agg_ref, deg_ref, wr_ref, wn_ref, b2_ref, o_ref):
  inv = 1.0 / jnp.maximum(deg_ref[0, :N] + deg_ref[1, :N], 1.0)
  mean2 = (agg_ref[0, :N] + agg_ref[1, :N]) * inv
  z = (jnp.dot(h_ref[:N], wr_ref[...], preferred_element_type=_f32)
       + jnp.dot(mean2, wn_ref[...], preferred_element_type=_f32)
       + b2_ref[...])
  z = z - jnp.max(z, axis=1, keepdims=True)
  o_ref[...] = z - jnp.log(jnp.sum(jnp.exp(z), axis=1, keepdims=True))


def kernel(x, edge_index, W1_root, W1_neigh, b1, W2_root, W2_neigh, b2):
  src = edge_index[0]
  dst = edge_index[1]
  pad = E_PAD - E
  src_p = jnp.concatenate([src, jnp.full((pad,), DUMMY, jnp.int32)])
  dst_p = jnp.concatenate([dst, jnp.full((pad,), DUMMY, jnp.int32)])
  src_p = src_p.reshape(NW * NCHUNK, CH)
  dst_p = dst_p.reshape(NW * NCHUNK, CH)
  zeros_c = jnp.zeros((CH, D_HID), _f32)
  ones_c = jnp.ones((CH, D_HID), _f32)

  p1, r1b = pl.pallas_call(
      _l1_proj_body,
      out_shape=[jax.ShapeDtypeStruct((N_PAD, D_HID), _f32),
                 jax.ShapeDtypeStruct((N_PAD, D_HID), _f32)],
  )(x, W1_neigh, W1_root, jnp.broadcast_to(b1, (1, D_HID)))

  agg1, deg = _sc_agg_deg(src_p, dst_p, p1, zeros_c, ones_c)

  agg2, h = _sc_pass2(src_p, dst_p, r1b, agg1, deg, zeros_c)
  agg2 = agg2.reshape(NC, N_PAD, D_HID)

  out = pl.pallas_call(
      _l2_out_body,
      out_shape=jax.ShapeDtypeStruct((N, D_OUT), _f32),
  )(h, agg2, deg.reshape(NC, N_PAD, D_HID), W2_root, W2_neigh,
    jnp.broadcast_to(b2, (1, D_OUT)))
  return out


# speedup vs baseline: 23.2059x; 1.1711x over previous
import functools

import jax
import jax.numpy as jnp
from jax import lax
from jax.experimental import pallas as pl
from jax.experimental.pallas import tpu as pltpu
from jax.experimental.pallas import tpu_sc as plsc

N = 10000
E = 320000
D_IN = 128
D_HID = 16
D_OUT = 40

NC = 2
NS = 16
NW = NC * NS
CH = 128
NCHUNK = 80
EPT = NCHUNK * CH
E_PAD = EPT * NW
N_PAD = 10240
RPC = N_PAD // NS
DUMMY = N_PAD - 1

_f32 = jnp.float32


NB = 4
HALF = N_PAD // NC


def _make_sc_agg(with_deg: bool):
  out_type = [jax.ShapeDtypeStruct((NC * N_PAD, D_HID), _f32)]
  if with_deg:
    out_type.append(jax.ShapeDtypeStruct((NC * N_PAD, D_HID), _f32))
  scratch = [
      pltpu.VMEM((NCHUNK, CH), jnp.int32),
      pltpu.VMEM((NCHUNK, CH), jnp.int32),
  ]
  scratch += [pltpu.VMEM((CH, D_HID), _f32) for _ in range(NB)]
  scratch += [
      pltpu.VMEM((CH, D_HID), _f32),
      pltpu.VMEM_SHARED((N_PAD, D_HID), _f32),
      pltpu.VMEM_SHARED((N_PAD, D_HID), _f32),
  ]
  scratch += [pltpu.SemaphoreType.DMA for _ in range(NB)]
  if with_deg:
    scratch.append(pltpu.VMEM((CH, D_HID), _f32))
    scratch.append(pltpu.VMEM_SHARED((N_PAD, D_HID), _f32))

  mesh = plsc.VectorSubcoreMesh(core_axis_name="c", subcore_axis_name="s")

  def fn(src_hbm, dst_hbm, feat_hbm, zeros_hbm, ones_hbm, *rest):
    if with_deg:
      agg_out, deg_out = rest[0], rest[1]
      sc = rest[2:]
    else:
      agg_out, deg_out = rest[0], None
      sc = rest[1:]
    src_v, dst_v = sc[0], sc[1]
    rows = sc[2:2 + NB]
    zeros_v = sc[2 + NB]
    agg_sh = sc[3 + NB]
    feat_sh = sc[4 + NB]
    gsems = sc[5 + NB:5 + 2 * NB]
    if with_deg:
      ones_v, deg_sh = sc[5 + 2 * NB], sc[6 + 2 * NB]

    c = lax.axis_index("c")
    s = lax.axis_index("s")
    wid = c * NS + s

    pltpu.sync_copy(src_hbm.at[pl.ds(wid * NCHUNK, NCHUNK)], src_v)
    pltpu.sync_copy(dst_hbm.at[pl.ds(wid * NCHUNK, NCHUNK)], dst_v)
    pltpu.sync_copy(zeros_hbm, zeros_v)
    if with_deg:
      pltpu.sync_copy(ones_hbm, ones_v)

    pltpu.sync_copy(feat_hbm.at[pl.ds(s * RPC, RPC)],
                    feat_sh.at[pl.ds(s * RPC, RPC)])

    for j in range(RPC // CH):
      off = s * RPC + j * CH
      pltpu.sync_copy(zeros_v, agg_sh.at[pl.ds(off, CH)])
      if with_deg:
        pltpu.sync_copy(zeros_v, deg_sh.at[pl.ds(off, CH)])
    plsc.subcore_barrier()

    for b in range(NB):
      pltpu.async_copy(feat_sh.at[src_v.at[b]], rows[b], gsems[b])

    @pl.loop(0, NCHUNK, step=NB)
    def _chunk_group(j0):
      for b in range(NB):
        j = j0 + b
        pltpu.make_async_copy(feat_sh.at[src_v.at[j]], rows[b],
                              gsems[b]).wait()
        pltpu.sync_copy(rows[b], agg_sh.at[dst_v.at[j]], add=True)
        if with_deg:
          pltpu.sync_copy(ones_v, deg_sh.at[dst_v.at[j]], add=True)

        @pl.when(j + NB < NCHUNK)
        def _prefetch():
          pltpu.async_copy(feat_sh.at[src_v.at[j + NB]], rows[b], gsems[b])

    plsc.subcore_barrier()

    dst_off = c * N_PAD + s * RPC
    pltpu.sync_copy(agg_sh.at[pl.ds(s * RPC, RPC)],
                    agg_out.at[pl.ds(dst_off, RPC)])
    if with_deg:
      pltpu.sync_copy(deg_sh.at[pl.ds(s * RPC, RPC)],
                      deg_out.at[pl.ds(dst_off, RPC)])

  return pl.kernel(
      fn, out_type=out_type, mesh=mesh, scratch_types=scratch,
      compiler_params=pltpu.CompilerParams(use_tc_tiling_on_sc=False))


_sc_agg_deg = _make_sc_agg(True)


def _make_sc_pass2():
  out_type = [
      jax.ShapeDtypeStruct((NC * N_PAD, D_HID), _f32),
      jax.ShapeDtypeStruct((N_PAD, D_HID), _f32),
  ]
  scratch = [
      pltpu.VMEM((NCHUNK, CH), jnp.int32),
      pltpu.VMEM((NCHUNK, CH), jnp.int32),
  ]
  scratch += [pltpu.VMEM((CH, D_HID), _f32) for _ in range(NB)]
  scratch += [
      pltpu.VMEM((CH, D_HID), _f32),
      pltpu.VMEM((RPC, D_HID), _f32),
      pltpu.VMEM((RPC, D_HID), _f32),
      pltpu.VMEM((RPC, D_HID), _f32),
      pltpu.VMEM((RPC, D_HID), _f32),
      pltpu.VMEM((RPC, D_HID), _f32),
      pltpu.VMEM_SHARED((N_PAD, D_HID), _f32),
      pltpu.VMEM_SHARED((N_PAD, D_HID), _f32),
  ]
  scratch += [pltpu.SemaphoreType.DMA for _ in range(NB)]

  mesh = plsc.VectorSubcoreMesh(core_axis_name="c", subcore_axis_name="s")

  def fn(src_hbm, dst_hbm, r1b_hbm, agg1_hbm, deg_hbm, zeros_hbm,
         agg2_out, h_out, src_v, dst_v, *sc):
    rows = sc[0:NB]
    zeros_v = sc[NB]
    hrow_v, a0_v, a1_v, d0_v, d1_v = sc[NB + 1:NB + 6]
    h_sh = sc[NB + 6]
    agg2_sh = sc[NB + 7]
    gsems = sc[NB + 8:NB + 8 + NB]

    c = lax.axis_index("c")
    s = lax.axis_index("s")
    wid = c * NS + s
    base = s * RPC

    pltpu.sync_copy(src_hbm.at[pl.ds(wid * NCHUNK, NCHUNK)], src_v)
    pltpu.sync_copy(dst_hbm.at[pl.ds(wid * NCHUNK, NCHUNK)], dst_v)
    pltpu.sync_copy(zeros_hbm, zeros_v)
    pltpu.sync_copy(r1b_hbm.at[pl.ds(base, RPC)], hrow_v)
    pltpu.sync_copy(agg1_hbm.at[pl.ds(base, RPC)], a0_v)
    pltpu.sync_copy(agg1_hbm.at[pl.ds(N_PAD + base, RPC)], a1_v)
    pltpu.sync_copy(deg_hbm.at[pl.ds(base, RPC)], d0_v)
    pltpu.sync_copy(deg_hbm.at[pl.ds(N_PAD + base, RPC)], d1_v)

    @pl.loop(0, RPC)
    def _hrow(i):
      mean1 = (a0_v[i] + a1_v[i]) / jnp.maximum(d0_v[i] + d1_v[i], 1.0)
      hrow_v[i] = jnp.maximum(hrow_v[i] + mean1, 0.0)

    pltpu.sync_copy(hrow_v, h_sh.at[pl.ds(base, RPC)])

    @pl.when(s // (NS // NC) == c)
    def _publish():
      pltpu.sync_copy(hrow_v, h_out.at[pl.ds(base, RPC)])

    for j in range(RPC // CH):
      pltpu.sync_copy(zeros_v, agg2_sh.at[pl.ds(base + j * CH, CH)])
    plsc.subcore_barrier()

    for b in range(NB):
      pltpu.async_copy(h_sh.at[src_v.at[b]], rows[b], gsems[b])

    @pl.loop(0, NCHUNK, step=NB)
    def _chunk_group(j0):
      for b in range(NB):
        j = j0 + b
        pltpu.make_async_copy(h_sh.at[src_v.at[j]], rows[b], gsems[b]).wait()
        pltpu.sync_copy(rows[b], agg2_sh.at[dst_v.at[j]], add=True)

        @pl.when(j + NB < NCHUNK)
        def _prefetch():
          pltpu.async_copy(h_sh.at[src_v.at[j + NB]], rows[b], gsems[b])

    plsc.subcore_barrier()
    pltpu.sync_copy(agg2_sh.at[pl.ds(base, RPC)],
                    agg2_out.at[pl.ds(c * N_PAD + base, RPC)])

  return pl.kernel(
      fn, out_type=out_type, mesh=mesh, scratch_types=scratch,
      compiler_params=pltpu.CompilerParams(use_tc_tiling_on_sc=False))


_sc_pass2 = _make_sc_pass2()


def _l1_proj_body(x_ref, wn_ref, wr_ref, b1_ref, p1_ref, r1b_ref):
  x = x_ref[...]
  p1_ref[:N] = jnp.dot(x, wn_ref[...], preferred_element_type=_f32)
  p1_ref[N:] = jnp.zeros((N_PAD - N, D_HID), _f32)
  r1b_ref[:N] = (jnp.dot(x, wr_ref[...], preferred_element_type=_f32)
                 + b1_ref[...])
  r1b_ref[N:] = jnp.zeros((N_PAD - N, D_HID), _f32)


def _l2_out_body(h_ref, agg_ref, deg_ref, wr_ref, wn_ref, b2_ref, o_ref):
  inv = 1.0 / jnp.maximum(deg_ref[0, :N] + deg_ref[1, :N], 1.0)
  mean2 = (agg_ref[0, :N] + agg_ref[1, :N]) * inv
  z = (jnp.dot(h_ref[:N], wr_ref[...], preferred_element_type=_f32)
       + jnp.dot(mean2, wn_ref[...], preferred_element_type=_f32)
       + b2_ref[...])
  z = z - jnp.max(z, axis=1, keepdims=True)
  o_ref[...] = z - jnp.log(jnp.sum(jnp.exp(z), axis=1, keepdims=True))


def kernel(x, edge_index, W1_root, W1_neigh, b1, W2_root, W2_neigh, b2):
  src = edge_index[0]
  dst = edge_index[1]
  pad = E_PAD - E
  src_p = jnp.concatenate([src, jnp.full((pad,), DUMMY, jnp.int32)])
  dst_p = jnp.concatenate([dst, jnp.full((pad,), DUMMY, jnp.int32)])
  src_p = src_p.reshape(NW * NCHUNK, CH)
  dst_p = dst_p.reshape(NW * NCHUNK, CH)
  zeros_c = jnp.zeros((CH, D_HID), _f32)
  ones_c = jnp.ones((CH, D_HID), _f32)

  p1, r1b = pl.pallas_call(
      _l1_proj_body,
      out_shape=[jax.ShapeDtypeStruct((N_PAD, D_HID), _f32),
                 jax.ShapeDtypeStruct((N_PAD, D_HID), _f32)],
  )(x, W1_neigh, W1_root, jnp.broadcast_to(b1, (1, D_HID)))

  agg1, deg = _sc_agg_deg(src_p, dst_p, p1, zeros_c, ones_c)

  agg2, h = _sc_pass2(src_p, dst_p, r1b, agg1, deg, zeros_c)
  agg2 = agg2.reshape(NC, N_PAD, D_HID)

  out = pl.pallas_call(
      _l2_out_body,
      out_shape=jax.ShapeDtypeStruct((N, D_OUT), _f32),
  )(h, agg2, deg.reshape(NC, N_PAD, D_HID), W2_root, W2_neigh,
    jnp.broadcast_to(b2, (1, D_OUT)))
  return out
